# baseline (speedup 1.0000x reference)
import functools

import jax
import jax.numpy as jnp
from jax import lax
from jax.experimental import pallas as pl
from jax.experimental.pallas import tpu as pltpu
from jax.experimental.pallas import tpu_sc as plsc

NUM_LAYER = 5
EMB = 128
N = 10000
E = 320000

NC = 2
NS = 16
NW = NC * NS
CH = 128
NP = 10240
NCH_E = 80
EPAD = NW * CH * NCH_E
NCH_N = NP // CH
RPT = NP // NS

_MESH = plsc.VectorSubcoreMesh(core_axis_name="c", subcore_axis_name="s")


def _sc_spmm(nch, width):
  stage = nch // 2

  @functools.partial(
      pl.kernel,
      out_type=jax.ShapeDtypeStruct((NC, NP, width), jnp.float32),
      mesh=_MESH,
      scratch_types=[
          pltpu.VMEM((stage, CH), jnp.int32),
          pltpu.VMEM((stage, CH), jnp.int32),
          pltpu.VMEM((CH, width), jnp.float32),
          pltpu.VMEM((CH, width), jnp.float32),
          pltpu.VMEM_SHARED((NP, width), jnp.float32),
          pltpu.SemaphoreType.DMA,
          pltpu.SemaphoreType.DMA,
      ],
  )
  def k(tbl_hbm, src_hbm, dst_hbm, zeros_hbm, out_hbm, srcv, dstv, rows0,
        rows1, acc, sem0, sem1):
    c = lax.axis_index("c")
    s = lax.axis_index("s")
    w = c * NS + s
    pltpu.sync_copy(zeros_hbm, acc.at[pl.ds(s * RPT, RPT)])
    plsc.subcore_barrier()

    for hh in range(2):
      pltpu.sync_copy(src_hbm.at[w, pl.ds(hh * stage, stage)], srcv)
      pltpu.sync_copy(dst_hbm.at[w, pl.ds(hh * stage, stage)], dstv)
      pltpu.async_copy(tbl_hbm.at[srcv.at[0]], rows0, sem0)

      def body(g, carry):
        j0 = 2 * g
        j1 = j0 + 1
        pltpu.async_copy(tbl_hbm.at[srcv.at[j1]], rows1, sem1)
        pltpu.make_async_copy(tbl_hbm.at[srcv.at[j0]], rows0, sem0).wait()
        pltpu.sync_copy(rows0, acc.at[dstv.at[j0]], add=True)

        @pl.when(j1 + 1 < stage)
        def _():
          pltpu.async_copy(tbl_hbm.at[srcv.at[j1 + 1]], rows0, sem0)

        pltpu.make_async_copy(tbl_hbm.at[srcv.at[j1]], rows1, sem1).wait()
        pltpu.sync_copy(rows1, acc.at[dstv.at[j1]], add=True)
        return carry

      lax.fori_loop(0, stage // 2, body, 0)

    plsc.subcore_barrier()
    pltpu.sync_copy(acc.at[pl.ds(s * RPT, RPT)],
                    out_hbm.at[c, pl.ds(s * RPT, RPT)])

  return k


def _sc_prepass(nch):
  half = nch // 2
  per_w = (NCH_N + NW - 1) // NW

  @functools.partial(
      pl.kernel,
      out_type=(jax.ShapeDtypeStruct((NP, EMB), jnp.float32),
                jax.ShapeDtypeStruct((NC, NP, EMB), jnp.float32)),
      mesh=_MESH,
      scratch_types=[
          pltpu.VMEM((CH,), jnp.int32),
          pltpu.VMEM((half, CH), jnp.int32),
          pltpu.VMEM((half, CH), jnp.int32),
          pltpu.VMEM((CH, EMB), jnp.float32),
          pltpu.VMEM((CH, EMB), jnp.float32),
          pltpu.VMEM_SHARED((NP, EMB), jnp.float32),
          pltpu.SemaphoreType.DMA,
          pltpu.SemaphoreType.DMA,
      ],
  )
  def k(etab_hbm, c0_hbm, onehot_hbm, cmb_hbm, dst_hbm, zeros_hbm,
        h_hbm, cnt_hbm, c0v, srcv, dstv, rows0, rows1, acc, sem0, sem1):
    c = lax.axis_index("c")
    s = lax.axis_index("s")
    w = c * NS + s
    pltpu.sync_copy(zeros_hbm, acc.at[pl.ds(s * RPT, RPT)])

    def ebody(t, carry):
      jj = w + t * NW

      @pl.when(jj < NCH_N)
      def _():
        pltpu.sync_copy(c0_hbm.at[jj], c0v)
        pltpu.async_copy(etab_hbm.at[c0v], rows0, sem0).wait()
        pltpu.sync_copy(rows0, h_hbm.at[pl.ds(jj * CH, CH)])

      return carry

    lax.fori_loop(0, per_w, ebody, 0)
    plsc.subcore_barrier()

    for hh in range(2):
      pltpu.sync_copy(cmb_hbm.at[w, pl.ds(hh * half, half)], srcv)
      pltpu.sync_copy(dst_hbm.at[w, pl.ds(hh * half, half)], dstv)
      pltpu.async_copy(onehot_hbm.at[srcv.at[0]], rows0, sem0)

      def body(g, carry):
        j0 = 2 * g
        j1 = j0 + 1
        pltpu.async_copy(onehot_hbm.at[srcv.at[j1]], rows1, sem1)
        pltpu.make_async_copy(onehot_hbm.at[srcv.at[j0]], rows0, sem0).wait()
        pltpu.sync_copy(rows0, acc.at[dstv.at[j0]], add=True)

        @pl.when(j1 + 1 < half)
        def _():
          pltpu.async_copy(onehot_hbm.at[srcv.at[j1 + 1]], rows0, sem0)

        pltpu.make_async_copy(onehot_hbm.at[srcv.at[j1]], rows1, sem1).wait()
        pltpu.sync_copy(rows1, acc.at[dstv.at[j1]], add=True)
        return carry

      lax.fori_loop(0, half // 2, body, 0)

    plsc.subcore_barrier()
    pltpu.sync_copy(acc.at[pl.ds(s * RPT, RPT)],
                    cnt_hbm.at[c, pl.ds(s * RPT, RPT)])

  return k


def _tc_layer(relu):
  def body(h_ref, pa_ref, pb_ref, ca_ref, cb_ref, T_ref, Wt_ref, s_ref,
           b2_ref, o_ref):
    agg = h_ref[...] + pa_ref[...] + pb_ref[...]
    cnt = ca_ref[...] + cb_ref[...]
    agg = agg + jnp.dot(cnt, T_ref[...], preferred_element_type=jnp.float32)
    y = jnp.dot(agg, Wt_ref[...], preferred_element_type=jnp.float32)
    y = y * s_ref[...] + b2_ref[...]
    if relu:
      y = jnp.maximum(y, 0.0)
    o_ref[...] = y

  return body


def _run_tc_layer(h, pa, pb, ca, cb, T, Wt, svec, b2, relu):
  blk = 1024
  grid = NP // blk
  row = lambda i: (i, 0)
  full = lambda i: (0, 0)
  return pl.pallas_call(
      _tc_layer(relu),
      grid=(grid,),
      in_specs=[
          pl.BlockSpec((blk, EMB), row),
          pl.BlockSpec((blk, EMB), row),
          pl.BlockSpec((blk, EMB), row),
          pl.BlockSpec((blk, 32), row),
          pl.BlockSpec((blk, 32), row),
          pl.BlockSpec((32, EMB), full),
          pl.BlockSpec((EMB, EMB), full),
          pl.BlockSpec((1, EMB), full),
          pl.BlockSpec((1, EMB), full),
      ],
      out_specs=pl.BlockSpec((blk, EMB), row),
      out_shape=jax.ShapeDtypeStruct((NP, EMB), jnp.float32),
  )(h, pa, pb, ca, cb, T, Wt, svec, b2)


def kernel(x, edge_index, edge_attr, Xemb1, Xemb2, W, b, Ee1, Ee2, gamma, beta):
  i32 = jnp.int32
  src = edge_index[0].astype(i32)
  dst = edge_index[1].astype(i32)
  combo = (edge_attr[:, 0] * 4 + edge_attr[:, 1]).astype(i32)

  pad = EPAD - E
  ar = jnp.arange(pad, dtype=i32)
  src_p = jnp.concatenate([src, ar % N]).reshape(NW, NCH_E, CH)
  dst_p = jnp.concatenate([dst, N + ar % (NP - N)]).reshape(NW, NCH_E, CH)
  cmb_p = jnp.concatenate([combo, jnp.zeros((pad,), i32)]).reshape(NW, NCH_E, CH)
  cmb_p = cmb_p + 32 * jnp.arange(CH, dtype=i32)[None, None, :]

  NCOMB = 480
  REP = 8
  etab = (Xemb1[:, None, :] + Xemb2[None, :, :]).reshape(NCOMB, EMB)
  etab = jnp.tile(etab, (REP, 1))
  c0 = x[:, 0].astype(i32) * 4 + x[:, 1].astype(i32)
  c0 = jnp.concatenate([c0, jnp.zeros((NP - N,), i32)])
  c0 = c0 + NCOMB * (jnp.arange(NP, dtype=i32) % REP)
  c0 = c0.reshape(NCH_N, CH)

  onehot = jnp.tile(jnp.eye(32, EMB, dtype=jnp.float32), (CH, 1))

  zeros_w = jnp.zeros((RPT, EMB), jnp.float32)

  cidx = jnp.arange(32)
  bn_inv = 1.0 / jnp.sqrt(1.0 + 1e-5)
  Ts, Wts, svecs, b2s = [], [], [], []
  for l in range(NUM_LAYER):
    T = Ee1[l][jnp.clip(cidx // 4, 0, 6)] + Ee2[l][cidx % 4]
    t16 = Ee1[l][4] + Ee2[l][0]
    Wt = W[l].T
    svec = (bn_inv * gamma[l]).reshape(1, EMB)
    b2 = ((b[l] + t16 @ Wt) * svec[0] + beta[l]).reshape(1, EMB)
    Ts.append(T)
    Wts.append(Wt)
    svecs.append(svec)
    b2s.append(b2)

  h, cacc = _sc_prepass(NCH_E)(etab, c0, onehot, cmb_p, dst_p, zeros_w)
  spmm = _sc_spmm(NCH_E, EMB)
  ca, cb = cacc[0, :, :32], cacc[1, :, :32]
  for l in range(NUM_LAYER):
    acc = spmm(h, src_p, dst_p, zeros_w)
    h = _run_tc_layer(h, acc[0], acc[1], ca, cb, Ts[l], Wts[l], svecs[l],
                      b2s[l], relu=l < NUM_LAYER - 1)

  return h[:N]


# speedup vs baseline: 1.0190x; 1.0190x over previous
import functools

import jax
import jax.numpy as jnp
from jax import lax
from jax.experimental import pallas as pl
from jax.experimental.pallas import tpu as pltpu
from jax.experimental.pallas import tpu_sc as plsc

NUM_LAYER = 5
EMB = 128
N = 10000
E = 320000

NC = 2
NS = 16
NW = NC * NS
CH = 128
NP = 10240
NCH_E = 80
EPAD = NW * CH * NCH_E
NCH_N = NP // CH
RPT = NP // NS

_MESH = plsc.VectorSubcoreMesh(core_axis_name="c", subcore_axis_name="s")


def _sc_spmm(nch, width):
  stage = nch // 2

  @functools.partial(
      pl.kernel,
      out_type=jax.ShapeDtypeStruct((NC, NP, width), jnp.float32),
      mesh=_MESH,
      scratch_types=[
          pltpu.VMEM((stage, CH), jnp.int32),
          pltpu.VMEM((stage, CH), jnp.int32),
          pltpu.VMEM((CH, width), jnp.float32),
          pltpu.VMEM((CH, width), jnp.float32),
          pltpu.VMEM_SHARED((NP, width), jnp.float32),
          pltpu.SemaphoreType.DMA,
          pltpu.SemaphoreType.DMA,
      ],
  )
  def k(tbl_hbm, src_hbm, dst_hbm, zeros_hbm, out_hbm, srcv, dstv, rows0,
        rows1, acc, sem0, sem1):
    c = lax.axis_index("c")
    s = lax.axis_index("s")
    w = c * NS + s
    pltpu.sync_copy(zeros_hbm, acc.at[pl.ds(s * RPT, RPT)])
    plsc.subcore_barrier()

    for hh in range(2):
      pltpu.sync_copy(src_hbm.at[w, pl.ds(hh * stage, stage)], srcv)
      pltpu.sync_copy(dst_hbm.at[w, pl.ds(hh * stage, stage)], dstv)
      pltpu.async_copy(tbl_hbm.at[srcv.at[0]], rows0, sem0)

      def body(g, carry):
        j0 = 2 * g
        j1 = j0 + 1
        pltpu.async_copy(tbl_hbm.at[srcv.at[j1]], rows1, sem1)
        pltpu.make_async_copy(tbl_hbm.at[srcv.at[j0]], rows0, sem0).wait()
        pltpu.sync_copy(rows0, acc.at[dstv.at[j0]], add=True)

        @pl.when(j1 + 1 < stage)
        def _():
          pltpu.async_copy(tbl_hbm.at[srcv.at[j1 + 1]], rows0, sem0)

        pltpu.make_async_copy(tbl_hbm.at[srcv.at[j1]], rows1, sem1).wait()
        pltpu.sync_copy(rows1, acc.at[dstv.at[j1]], add=True)
        return carry

      lax.fori_loop(0, stage // 2, body, 0)

    plsc.subcore_barrier()
    pltpu.sync_copy(acc.at[pl.ds(s * RPT, RPT)],
                    out_hbm.at[c, pl.ds(s * RPT, RPT)])

  return k


def _sc_embed():
  per_w = (NCH_N + NW - 1) // NW

  @functools.partial(
      pl.kernel,
      out_type=jax.ShapeDtypeStruct((NP, EMB), jnp.float32),
      mesh=_MESH,
      scratch_types=[
          pltpu.VMEM((CH,), jnp.int32),
          pltpu.VMEM((CH, EMB), jnp.float32),
          pltpu.SemaphoreType.DMA,
      ],
  )
  def k(tbl_hbm, c0_hbm, out_hbm, c0v, rows, sem):
    c = lax.axis_index("c")
    s = lax.axis_index("s")
    w = c * NS + s

    def body(t, carry):
      jj = w + t * NW

      @pl.when(jj < NCH_N)
      def _():
        pltpu.sync_copy(c0_hbm.at[jj], c0v)
        pltpu.async_copy(tbl_hbm.at[c0v], rows, sem).wait()
        pltpu.sync_copy(rows, out_hbm.at[pl.ds(jj * CH, CH)])

      return carry

    lax.fori_loop(0, per_w, body, 0)

  return k


def _tc_layer(relu):
  def body(h_ref, pa_ref, pb_ref, ca_ref, cb_ref, T_ref, Wt_ref, s_ref,
           b2_ref, o_ref):
    agg = h_ref[...] + pa_ref[...] + pb_ref[...]
    cnt = ca_ref[...] + cb_ref[...]
    agg = agg + jnp.dot(cnt, T_ref[...], preferred_element_type=jnp.float32)
    y = jnp.dot(agg, Wt_ref[...], preferred_element_type=jnp.float32)
    y = y * s_ref[...] + b2_ref[...]
    if relu:
      y = jnp.maximum(y, 0.0)
    o_ref[...] = y

  return body


def _run_tc_layer(h, pa, pb, ca, cb, T, Wt, svec, b2, relu):
  blk = 1024
  grid = NP // blk
  row = lambda i: (i, 0)
  full = lambda i: (0, 0)
  return pl.pallas_call(
      _tc_layer(relu),
      grid=(grid,),
      in_specs=[
          pl.BlockSpec((blk, EMB), row),
          pl.BlockSpec((blk, EMB), row),
          pl.BlockSpec((blk, EMB), row),
          pl.BlockSpec((blk, 32), row),
          pl.BlockSpec((blk, 32), row),
          pl.BlockSpec((32, EMB), full),
          pl.BlockSpec((EMB, EMB), full),
          pl.BlockSpec((1, EMB), full),
          pl.BlockSpec((1, EMB), full),
      ],
      out_specs=pl.BlockSpec((blk, EMB), row),
      out_shape=jax.ShapeDtypeStruct((NP, EMB), jnp.float32),
  )(h, pa, pb, ca, cb, T, Wt, svec, b2)


def kernel(x, edge_index, edge_attr, Xemb1, Xemb2, W, b, Ee1, Ee2, gamma, beta):
  i32 = jnp.int32
  src = edge_index[0].astype(i32)
  dst = edge_index[1].astype(i32)
  combo = (edge_attr[:, 0] * 4 + edge_attr[:, 1]).astype(i32)

  pad = EPAD - E
  ar = jnp.arange(pad, dtype=i32)
  src_p = jnp.concatenate([src, ar % N]).reshape(NW, NCH_E, CH)
  dst_p = jnp.concatenate([dst, N + ar % (NP - N)]).reshape(NW, NCH_E, CH)
  cmb_p = jnp.concatenate([combo, jnp.zeros((pad,), i32)]).reshape(NW, NCH_E, CH)
  cmb_p = cmb_p + 32 * jnp.arange(CH, dtype=i32)[None, None, :]

  NCOMB = 480
  REP = 8
  etab = (Xemb1[:, None, :] + Xemb2[None, :, :]).reshape(NCOMB, EMB)
  etab = jnp.tile(etab, (REP, 1))
  c0 = x[:, 0].astype(i32) * 4 + x[:, 1].astype(i32)
  c0 = jnp.concatenate([c0, jnp.zeros((NP - N,), i32)])
  c0 = c0 + NCOMB * (jnp.arange(NP, dtype=i32) % REP)
  c0 = c0.reshape(NCH_N, CH)

  onehot = jnp.tile(jnp.eye(32, EMB, dtype=jnp.float32), (CH, 1))

  zeros_w = jnp.zeros((RPT, EMB), jnp.float32)

  cidx = jnp.arange(32)
  bn_inv = 1.0 / jnp.sqrt(1.0 + 1e-5)
  Ts, Wts, svecs, b2s = [], [], [], []
  for l in range(NUM_LAYER):
    T = Ee1[l][jnp.clip(cidx // 4, 0, 6)] + Ee2[l][cidx % 4]
    t16 = Ee1[l][4] + Ee2[l][0]
    Wt = W[l].T
    svec = (bn_inv * gamma[l]).reshape(1, EMB)
    b2 = ((b[l] + t16 @ Wt) * svec[0] + beta[l]).reshape(1, EMB)
    Ts.append(T)
    Wts.append(Wt)
    svecs.append(svec)
    b2s.append(b2)

  h = _sc_embed()(etab, c0)
  spmm = _sc_spmm(NCH_E, EMB)
  cacc = spmm(onehot, cmb_p, dst_p, zeros_w)
  ca, cb = cacc[0, :, :32], cacc[1, :, :32]
  for l in range(NUM_LAYER):
    acc = spmm(h, src_p, dst_p, zeros_w)
    h = _run_tc_layer(h, acc[0], acc[1], ca, cb, Ts[l], Wts[l], svecs[l],
                      b2s[l], relu=l < NUM_LAYER - 1)

  return h[:N]
